# Initial kernel scaffold; baseline (speedup 1.0000x reference)
import jax
import jax.numpy as jnp
from jax.experimental import pallas as pl


def kernel(item, locs, times, session_emb, params, Hg):
    raise NotImplementedError("write your pallas kernel here")


# speedup vs baseline: 5.6927x; 5.6927x over previous
import functools

import jax
import jax.numpy as jnp
from jax import lax
from jax.experimental import pallas as pl
from jax.experimental.pallas import tpu as pltpu
from jax.experimental.pallas import tpu_sc as plsc

D = 64
K = 16
CHUNK = 32
FLAT = CHUNK * K
SUB = 128
NC = 2
NS = 16
NW = NC * NS


IDXC = 80


def _mesh():
    return plsc.VectorSubcoreMesh(
        core_axis_name="c", subcore_axis_name="s", num_cores=NC,
        num_subcores=NS)


def _sc_gather_hg(iids, locs, times, hg):
    nb = iids.shape[0]
    ns = locs.shape[0]
    i32 = jnp.int32
    out_type = tuple(
        jax.ShapeDtypeStruct(s, i32)
        for s in [(nb, K), (nb, K), (nb, K),
                  (ns, K), (ns, K), (ns, K), (ns, K)]
    )
    scratch = [
        pltpu.VMEM((IDXC,), i32),
        pltpu.VMEM((IDXC, K), i32),
        pltpu.VMEM((IDXC, K), i32),
        pltpu.VMEM((IDXC, K), i32),
    ]

    @functools.partial(
        pl.kernel, out_type=out_type, mesh=_mesh(), scratch_types=scratch,
        compiler_params=pltpu.CompilerParams(use_tc_tiling_on_sc=False))
    def s1(iids_h, locs_h, times_h, h_ii, h_ti, h_li, h_il, h_tl, h_it, h_lt,
           o_ii, o_ti, o_li, o_il, o_tl, o_it, o_lt,
           idx_v, nbr0, nbr1, nbr2):
        wid = lax.axis_index("s") * NC + lax.axis_index("c")

        def grab(idx_hbm, base, c, rels):
            pltpu.sync_copy(idx_hbm.at[pl.ds(base, c)], idx_v.at[pl.ds(0, c)])
            for hgt, nbr, o in rels:
                pltpu.sync_copy(hgt.at[idx_v.at[pl.ds(0, c)]],
                                nbr.at[pl.ds(0, c)])
                pltpu.sync_copy(nbr.at[pl.ds(0, c)], o.at[pl.ds(base, c)])

        rows_w = nb // NW

        def big(j, carry):
            base = wid * rows_w + j * IDXC
            grab(iids_h, base, IDXC,
                 [(h_ii, nbr0, o_ii), (h_ti, nbr1, o_ti), (h_li, nbr2, o_li)])
            return carry
        lax.fori_loop(0, rows_w // IDXC, big, 0)

        base_s = wid * (ns // NW)
        grab(locs_h, base_s, ns // NW,
             [(h_il, nbr0, o_il), (h_tl, nbr1, o_tl)])
        grab(times_h, base_s, ns // NW,
             [(h_it, nbr0, o_it), (h_lt, nbr1, o_lt)])

    return s1(iids, locs, times, hg['II'], hg['TI'], hg['LI'], hg['IL'],
              hg['TL'], hg['IT'], hg['LT'])


def _sc_gather_means(item_tab, loc_tab, time_tab, iids, locs, times,
                     f_ii, f_ti, f_li, f_il, f_tl, f_it, f_lt):
    nb = iids.shape[0]
    ns = locs.shape[0]
    rows_w = nb // NW
    n_chunks = rows_w // CHUNK
    srows_w = ns // NW
    f32 = jnp.float32

    out_type = tuple(
        jax.ShapeDtypeStruct(s, f32)
        for s in [(nb, D), (nb, D), (nb, D), (nb, D),
                  (ns, D), (ns, D), (ns, D),
                  (ns, D), (ns, D), (ns, D)]
    )
    scratch = [
        pltpu.VMEM((IDXC,), jnp.int32),
        pltpu.VMEM((IDXC, D), f32),
        pltpu.VMEM((FLAT,), jnp.int32),
        pltpu.VMEM((FLAT, D), f32),
        pltpu.VMEM((CHUNK, D), f32),
    ]

    @functools.partial(
        pl.kernel, out_type=out_type, mesh=_mesh(), scratch_types=scratch,
        compiler_params=pltpu.CompilerParams(use_tc_tiling_on_sc=False))
    def s2(item_t, loc_t, time_t, iids_h, locs_h, times_h,
           fii, fti, fli, fil, ftl, fit, flt,
           o_self_i, o_mii, o_mti, o_mli,
           o_self_l, o_mil, o_mtl,
           o_self_t, o_mit, o_mlt,
           sidx, srows, fidx, rows, mbuf):
        wid = lax.axis_index("s") * NC + lax.axis_index("c")

        def reduce_mean(c_rows):
            def body(c, carry):
                r0 = c * K
                for d in range(D // 16):
                    sl = pl.ds(d * 16, 16)
                    acc = rows[r0, sl]
                    for kk in range(1, K):
                        acc = acc + rows[r0 + kk, sl]
                    mbuf[c, sl] = acc * (1.0 / K)
                return carry
            lax.fori_loop(0, c_rows, body, 0)

        def self_job(idx_hbm, n, tab, out):
            per_w = n // NW

            def one(j, carry):
                c = min(IDXC, per_w)
                base = wid * per_w + j * c
                pltpu.sync_copy(idx_hbm.at[pl.ds(base, c)],
                                sidx.at[pl.ds(0, c)])
                pltpu.sync_copy(tab.at[sidx.at[pl.ds(0, c)]],
                                srows.at[pl.ds(0, c)])
                pltpu.sync_copy(srows.at[pl.ds(0, c)],
                                out.at[pl.ds(base, c)])
                return carry
            lax.fori_loop(0, max(1, per_w // IDXC), one, 0)

        def mean_job(flat_hbm, n, tab, out):
            per_w = n // NW
            c_rows = min(CHUNK, per_w)
            nfl = c_rows * K

            def one(j, carry):
                base = wid * per_w + j * c_rows
                fbase = base * K
                pltpu.sync_copy(flat_hbm.at[pl.ds(fbase, nfl)],
                                fidx.at[pl.ds(0, nfl)])
                for t in range(nfl // SUB):
                    pltpu.sync_copy(
                        tab.at[fidx.at[pl.ds(t * SUB, SUB)]],
                        rows.at[pl.ds(t * SUB, SUB)])
                reduce_mean(c_rows)
                pltpu.sync_copy(mbuf.at[pl.ds(0, c_rows)],
                                out.at[pl.ds(base, c_rows)])
                return carry
            lax.fori_loop(0, max(1, per_w // c_rows), one, 0)

        self_job(iids_h, nb, item_t, o_self_i)
        self_job(locs_h, ns, loc_t, o_self_l)
        self_job(times_h, ns, time_t, o_self_t)
        mean_job(fii, nb, item_t, o_mii)
        mean_job(fti, nb, time_t, o_mti)
        mean_job(fli, nb, loc_t, o_mli)
        mean_job(fil, ns, item_t, o_mil)
        mean_job(ftl, ns, time_t, o_mtl)
        mean_job(fit, ns, item_t, o_mit)
        mean_job(flt, ns, loc_t, o_mlt)

    return s2(item_tab, loc_tab, time_tab, iids, locs, times,
              f_ii, f_ti, f_li, f_il, f_tl, f_it, f_lt)


def _att_head(self_rows, aggs, ws, bs, sess, att_w1, att_b1, att_q):
    n, d = self_rows.shape
    n_agg = len(aggs)
    m = n_agg + (1 if sess is not None else 0)
    r = min(512, n)
    g = n // r
    f32 = jnp.float32
    inv_n = 1.0 / n

    row_spec = pl.BlockSpec((r, d), lambda i: (i, 0))
    w_spec = pl.BlockSpec((2 * d, d), lambda i: (0, 0))
    b_spec = pl.BlockSpec((1, d), lambda i: (0, 0))
    w1_spec = pl.BlockSpec((d, d), lambda i: (0, 0))
    q_spec = pl.BlockSpec((d, 1), lambda i: (0, 0))
    smem_spec = pl.BlockSpec(memory_space=pltpu.SMEM)

    sess_in = [sess] if sess is not None else []
    sess_specs = [row_spec] if sess is not None else []

    def heads(self_b, agg_b, sess_b, w_b, b_b):
        hs = []
        for p in range(n_agg):
            h = (jnp.dot(self_b, w_b[p][:d, :], preferred_element_type=f32)
                 + jnp.dot(agg_b[p], w_b[p][d:, :], preferred_element_type=f32)
                 + b_b[p])
            hs.append(jnp.maximum(h, 0.0))
        if sess_b is not None:
            hs.append(sess_b)
        return hs

    def body_a(*args):
        i = pl.program_id(0)
        it = iter(args)
        self_b = next(it)[...]
        agg_b = [next(it)[...] for _ in range(n_agg)]
        sess_b = next(it)[...] if sess is not None else None
        w_b = [next(it)[...] for _ in range(n_agg)]
        b_b = [next(it)[...] for _ in range(n_agg)]
        w1_b = next(it)[...]
        b1_b = next(it)[...]
        q_b = next(it)[...]
        wsum_ref = next(it)
        hs = heads(self_b, agg_b, sess_b, w_b, b_b)

        @pl.when(i == 0)
        def _():
            for p in range(m):
                wsum_ref[p] = 0.0

        for p in range(m):
            w = jnp.dot(
                jnp.tanh(jnp.dot(hs[p], w1_b, preferred_element_type=f32)
                         + b1_b),
                q_b, preferred_element_type=f32)
            wsum_ref[p] += jnp.sum(w)

    wsum = pl.pallas_call(
        body_a,
        grid=(g,),
        in_specs=[row_spec] + [row_spec] * n_agg + sess_specs
                 + [w_spec] * n_agg + [b_spec] * n_agg
                 + [w1_spec, b_spec, q_spec],
        out_specs=smem_spec,
        out_shape=jax.ShapeDtypeStruct((m,), f32),
    )(self_rows, *aggs, *sess_in, *ws, *bs, att_w1, att_b1, att_q)

    def body_b(*args):
        it = iter(args)
        self_b = next(it)[...]
        agg_b = [next(it)[...] for _ in range(n_agg)]
        sess_b = next(it)[...] if sess is not None else None
        w_b = [next(it)[...] for _ in range(n_agg)]
        b_b = [next(it)[...] for _ in range(n_agg)]
        wsum_ref = next(it)
        out_ref = next(it)
        hs = heads(self_b, agg_b, sess_b, w_b, b_b)

        sv = [jnp.full((1, 1), wsum_ref[p] * inv_n, f32) for p in range(m)]
        mx = sv[0]
        for p in range(1, m):
            mx = jnp.maximum(mx, sv[p])
        ev = [jnp.exp(sv[p] - mx) for p in range(m)]
        denom = ev[0]
        for p in range(1, m):
            denom = denom + ev[p]
        acc = hs[0] * ev[0]
        for p in range(1, m):
            acc = acc + hs[p] * ev[p]
        out_ref[...] = acc / denom

    return pl.pallas_call(
        body_b,
        grid=(g,),
        in_specs=[row_spec] + [row_spec] * n_agg + sess_specs
                 + [w_spec] * n_agg + [b_spec] * n_agg + [smem_spec],
        out_specs=row_spec,
        out_shape=jax.ShapeDtypeStruct((n, d), f32),
    )(self_rows, *aggs, *sess_in, *ws, *bs, wsum)


def kernel(item, locs, times, session_emb, params, Hg):
    iids = item.reshape(-1)
    hg_rows = _sc_gather_hg(iids, locs, times, Hg)
    flats = tuple(a.reshape(-1) for a in hg_rows)
    (self_i, mii, mti, mli, self_l, mil, mtl, self_t, mit, mlt) = \
        _sc_gather_means(params['item_tab'], params['loc_tab'],
                         params['time_tab'], iids, locs, times, *flats)
    sess = session_emb[:, 0, :]

    def b2(name):
        return params[name].reshape(1, D)

    h_items = _att_head(
        self_i, [mii, mti, mli],
        [params['II_W'], params['TI_W'], params['LI_W']],
        [b2('II_b'), b2('TI_b'), b2('LI_b')],
        None, params['att_i_W1'], b2('att_i_b1'), params['att_i_q'])
    h_locs = _att_head(
        self_l, [mil, mtl],
        [params['IL_W'], params['TL_W']],
        [b2('IL_b'), b2('TL_b')],
        sess, params['att_l_W1'], b2('att_l_b1'), params['att_l_q'])
    h_times = _att_head(
        self_t, [mit, mlt],
        [params['IT_W'], params['LT_W']],
        [b2('IT_b'), b2('LT_b')],
        sess, params['att_t_W1'], b2('att_t_b1'), params['att_t_q'])
    return (h_items, h_locs, h_times)


# speedup vs baseline: 6.8355x; 1.2007x over previous
import functools

import jax
import jax.numpy as jnp
from jax import lax
from jax.experimental import pallas as pl
from jax.experimental.pallas import tpu as pltpu
from jax.experimental.pallas import tpu_sc as plsc

D = 64
K = 16
CHUNK = 32
FLAT = CHUNK * K
SUB = 512
NC = 2
NS = 16
NW = NC * NS


IDXC = 80


def _mesh():
    return plsc.VectorSubcoreMesh(
        core_axis_name="c", subcore_axis_name="s", num_cores=NC,
        num_subcores=NS)


def _sc_gather_hg(iids, locs, times, hg):
    nb = iids.shape[0]
    ns = locs.shape[0]
    i32 = jnp.int32
    out_type = tuple(
        jax.ShapeDtypeStruct(s, i32)
        for s in [(nb, K), (nb, K), (nb, K),
                  (ns, K), (ns, K), (ns, K), (ns, K)]
    )
    scratch = [
        pltpu.VMEM((IDXC,), i32),
        pltpu.VMEM((IDXC, K), i32),
        pltpu.VMEM((IDXC, K), i32),
        pltpu.VMEM((IDXC, K), i32),
    ]

    @functools.partial(
        pl.kernel, out_type=out_type, mesh=_mesh(), scratch_types=scratch,
        compiler_params=pltpu.CompilerParams(use_tc_tiling_on_sc=False))
    def s1(iids_h, locs_h, times_h, h_ii, h_ti, h_li, h_il, h_tl, h_it, h_lt,
           o_ii, o_ti, o_li, o_il, o_tl, o_it, o_lt,
           idx_v, nbr0, nbr1, nbr2):
        wid = lax.axis_index("s") * NC + lax.axis_index("c")

        def grab(idx_hbm, base, c, rels):
            pltpu.sync_copy(idx_hbm.at[pl.ds(base, c)], idx_v.at[pl.ds(0, c)])
            for hgt, nbr, o in rels:
                pltpu.sync_copy(hgt.at[idx_v.at[pl.ds(0, c)]],
                                nbr.at[pl.ds(0, c)])
                pltpu.sync_copy(nbr.at[pl.ds(0, c)], o.at[pl.ds(base, c)])

        rows_w = nb // NW

        def big(j, carry):
            base = wid * rows_w + j * IDXC
            grab(iids_h, base, IDXC,
                 [(h_ii, nbr0, o_ii), (h_ti, nbr1, o_ti), (h_li, nbr2, o_li)])
            return carry
        lax.fori_loop(0, rows_w // IDXC, big, 0)

        base_s = wid * (ns // NW)
        grab(locs_h, base_s, ns // NW,
             [(h_il, nbr0, o_il), (h_tl, nbr1, o_tl)])
        grab(times_h, base_s, ns // NW,
             [(h_it, nbr0, o_it), (h_lt, nbr1, o_lt)])

    return s1(iids, locs, times, hg['II'], hg['TI'], hg['LI'], hg['IL'],
              hg['TL'], hg['IT'], hg['LT'])


def _sc_gather_means(item_tab, loc_tab, time_tab, iids, locs, times,
                     f_ii, f_ti, f_li, f_il, f_tl, f_it, f_lt):
    nb = iids.shape[0]
    ns = locs.shape[0]
    rows_w = nb // NW
    n_chunks = rows_w // CHUNK
    srows_w = ns // NW
    f32 = jnp.float32

    out_type = tuple(
        jax.ShapeDtypeStruct(s, f32)
        for s in [(nb, D), (nb, D), (nb, D), (nb, D),
                  (ns, D), (ns, D), (ns, D),
                  (ns, D), (ns, D), (ns, D)]
    )
    scratch = [
        pltpu.VMEM((IDXC,), jnp.int32),
        pltpu.VMEM((IDXC, D), f32),
        pltpu.VMEM((FLAT,), jnp.int32),
        pltpu.VMEM((FLAT, D), f32),
        pltpu.VMEM((CHUNK, D), f32),
    ]

    @functools.partial(
        pl.kernel, out_type=out_type, mesh=_mesh(), scratch_types=scratch,
        compiler_params=pltpu.CompilerParams(use_tc_tiling_on_sc=False))
    def s2(item_t, loc_t, time_t, iids_h, locs_h, times_h,
           fii, fti, fli, fil, ftl, fit, flt,
           o_self_i, o_mii, o_mti, o_mli,
           o_self_l, o_mil, o_mtl,
           o_self_t, o_mit, o_mlt,
           sidx, srows, fidx, rows, mbuf):
        wid = lax.axis_index("s") * NC + lax.axis_index("c")

        def reduce_mean(c_rows):
            def body(c, carry):
                r0 = c * K
                for d in range(D // 16):
                    sl = pl.ds(d * 16, 16)
                    acc = rows[r0, sl]
                    for kk in range(1, K):
                        acc = acc + rows[r0 + kk, sl]
                    mbuf[c, sl] = acc * (1.0 / K)
                return carry
            lax.fori_loop(0, c_rows, body, 0)

        def self_job(idx_hbm, n, tab, out):
            per_w = n // NW

            def one(j, carry):
                c = min(IDXC, per_w)
                base = wid * per_w + j * c
                pltpu.sync_copy(idx_hbm.at[pl.ds(base, c)],
                                sidx.at[pl.ds(0, c)])
                pltpu.sync_copy(tab.at[sidx.at[pl.ds(0, c)]],
                                srows.at[pl.ds(0, c)])
                pltpu.sync_copy(srows.at[pl.ds(0, c)],
                                out.at[pl.ds(base, c)])
                return carry
            lax.fori_loop(0, max(1, per_w // IDXC), one, 0)

        def mean_job(flat_hbm, n, tab, out):
            per_w = n // NW
            c_rows = min(CHUNK, per_w)
            nfl = c_rows * K

            def one(j, carry):
                base = wid * per_w + j * c_rows
                fbase = base * K
                pltpu.sync_copy(flat_hbm.at[pl.ds(fbase, nfl)],
                                fidx.at[pl.ds(0, nfl)])
                for t in range(nfl // SUB):
                    pltpu.sync_copy(
                        tab.at[fidx.at[pl.ds(t * SUB, SUB)]],
                        rows.at[pl.ds(t * SUB, SUB)])
                reduce_mean(c_rows)
                pltpu.sync_copy(mbuf.at[pl.ds(0, c_rows)],
                                out.at[pl.ds(base, c_rows)])
                return carry
            lax.fori_loop(0, max(1, per_w // c_rows), one, 0)

        self_job(iids_h, nb, item_t, o_self_i)
        self_job(locs_h, ns, loc_t, o_self_l)
        self_job(times_h, ns, time_t, o_self_t)
        mean_job(fii, nb, item_t, o_mii)
        mean_job(fti, nb, time_t, o_mti)
        mean_job(fli, nb, loc_t, o_mli)
        mean_job(fil, ns, item_t, o_mil)
        mean_job(ftl, ns, time_t, o_mtl)
        mean_job(fit, ns, item_t, o_mit)
        mean_job(flt, ns, loc_t, o_mlt)

    return s2(item_tab, loc_tab, time_tab, iids, locs, times,
              f_ii, f_ti, f_li, f_il, f_tl, f_it, f_lt)


def _att_head(self_rows, aggs, ws, bs, sess, att_w1, att_b1, att_q):
    n, d = self_rows.shape
    n_agg = len(aggs)
    m = n_agg + (1 if sess is not None else 0)
    r = min(512, n)
    g = n // r
    f32 = jnp.float32
    inv_n = 1.0 / n

    row_spec = pl.BlockSpec((r, d), lambda i: (i, 0))
    w_spec = pl.BlockSpec((2 * d, d), lambda i: (0, 0))
    b_spec = pl.BlockSpec((1, d), lambda i: (0, 0))
    w1_spec = pl.BlockSpec((d, d), lambda i: (0, 0))
    q_spec = pl.BlockSpec((d, 1), lambda i: (0, 0))
    smem_spec = pl.BlockSpec(memory_space=pltpu.SMEM)

    sess_in = [sess] if sess is not None else []
    sess_specs = [row_spec] if sess is not None else []

    def heads(self_b, agg_b, sess_b, w_b, b_b):
        hs = []
        for p in range(n_agg):
            h = (jnp.dot(self_b, w_b[p][:d, :], preferred_element_type=f32)
                 + jnp.dot(agg_b[p], w_b[p][d:, :], preferred_element_type=f32)
                 + b_b[p])
            hs.append(jnp.maximum(h, 0.0))
        if sess_b is not None:
            hs.append(sess_b)
        return hs

    def body_a(*args):
        i = pl.program_id(0)
        it = iter(args)
        self_b = next(it)[...]
        agg_b = [next(it)[...] for _ in range(n_agg)]
        sess_b = next(it)[...] if sess is not None else None
        w_b = [next(it)[...] for _ in range(n_agg)]
        b_b = [next(it)[...] for _ in range(n_agg)]
        w1_b = next(it)[...]
        b1_b = next(it)[...]
        q_b = next(it)[...]
        wsum_ref = next(it)
        hs = heads(self_b, agg_b, sess_b, w_b, b_b)

        @pl.when(i == 0)
        def _():
            for p in range(m):
                wsum_ref[p] = 0.0

        for p in range(m):
            w = jnp.dot(
                jnp.tanh(jnp.dot(hs[p], w1_b, preferred_element_type=f32)
                         + b1_b),
                q_b, preferred_element_type=f32)
            wsum_ref[p] += jnp.sum(w)

    wsum = pl.pallas_call(
        body_a,
        grid=(g,),
        in_specs=[row_spec] + [row_spec] * n_agg + sess_specs
                 + [w_spec] * n_agg + [b_spec] * n_agg
                 + [w1_spec, b_spec, q_spec],
        out_specs=smem_spec,
        out_shape=jax.ShapeDtypeStruct((m,), f32),
    )(self_rows, *aggs, *sess_in, *ws, *bs, att_w1, att_b1, att_q)

    def body_b(*args):
        it = iter(args)
        self_b = next(it)[...]
        agg_b = [next(it)[...] for _ in range(n_agg)]
        sess_b = next(it)[...] if sess is not None else None
        w_b = [next(it)[...] for _ in range(n_agg)]
        b_b = [next(it)[...] for _ in range(n_agg)]
        wsum_ref = next(it)
        out_ref = next(it)
        hs = heads(self_b, agg_b, sess_b, w_b, b_b)

        sv = [jnp.full((1, 1), wsum_ref[p] * inv_n, f32) for p in range(m)]
        mx = sv[0]
        for p in range(1, m):
            mx = jnp.maximum(mx, sv[p])
        ev = [jnp.exp(sv[p] - mx) for p in range(m)]
        denom = ev[0]
        for p in range(1, m):
            denom = denom + ev[p]
        acc = hs[0] * ev[0]
        for p in range(1, m):
            acc = acc + hs[p] * ev[p]
        out_ref[...] = acc / denom

    return pl.pallas_call(
        body_b,
        grid=(g,),
        in_specs=[row_spec] + [row_spec] * n_agg + sess_specs
                 + [w_spec] * n_agg + [b_spec] * n_agg + [smem_spec],
        out_specs=row_spec,
        out_shape=jax.ShapeDtypeStruct((n, d), f32),
    )(self_rows, *aggs, *sess_in, *ws, *bs, wsum)


def kernel(item, locs, times, session_emb, params, Hg):
    iids = item.reshape(-1)
    hg_rows = _sc_gather_hg(iids, locs, times, Hg)
    flats = tuple(a.reshape(-1) for a in hg_rows)
    (self_i, mii, mti, mli, self_l, mil, mtl, self_t, mit, mlt) = \
        _sc_gather_means(params['item_tab'], params['loc_tab'],
                         params['time_tab'], iids, locs, times, *flats)
    sess = session_emb[:, 0, :]

    def b2(name):
        return params[name].reshape(1, D)

    h_items = _att_head(
        self_i, [mii, mti, mli],
        [params['II_W'], params['TI_W'], params['LI_W']],
        [b2('II_b'), b2('TI_b'), b2('LI_b')],
        None, params['att_i_W1'], b2('att_i_b1'), params['att_i_q'])
    h_locs = _att_head(
        self_l, [mil, mtl],
        [params['IL_W'], params['TL_W']],
        [b2('IL_b'), b2('TL_b')],
        sess, params['att_l_W1'], b2('att_l_b1'), params['att_l_q'])
    h_times = _att_head(
        self_t, [mit, mlt],
        [params['IT_W'], params['LT_W']],
        [b2('IT_b'), b2('LT_b')],
        sess, params['att_t_W1'], b2('att_t_b1'), params['att_t_q'])
    return (h_items, h_locs, h_times)


# speedup vs baseline: 8.4012x; 1.2291x over previous
import functools

import jax
import jax.numpy as jnp
from jax import lax
from jax.experimental import pallas as pl
from jax.experimental.pallas import tpu as pltpu
from jax.experimental.pallas import tpu_sc as plsc

D = 64
K = 16
CHUNK = 32
FLAT = CHUNK * K
SUB = 512
NC = 2
NS = 16
NW = NC * NS


IDXC = 80


def _mesh():
    return plsc.VectorSubcoreMesh(
        core_axis_name="c", subcore_axis_name="s", num_cores=NC,
        num_subcores=NS)


def _sc_gather_hg(iids, locs, times, hg):
    nb = iids.shape[0]
    ns = locs.shape[0]
    i32 = jnp.int32
    out_type = tuple(
        jax.ShapeDtypeStruct(s, i32)
        for s in [(nb, K), (nb, K), (nb, K),
                  (ns, K), (ns, K), (ns, K), (ns, K)]
    )
    scratch = [
        pltpu.VMEM((IDXC,), i32),
        pltpu.VMEM((IDXC, K), i32),
        pltpu.VMEM((IDXC, K), i32),
        pltpu.VMEM((IDXC, K), i32),
    ]

    @functools.partial(
        pl.kernel, out_type=out_type, mesh=_mesh(), scratch_types=scratch,
        compiler_params=pltpu.CompilerParams(use_tc_tiling_on_sc=False))
    def s1(iids_h, locs_h, times_h, h_ii, h_ti, h_li, h_il, h_tl, h_it, h_lt,
           o_ii, o_ti, o_li, o_il, o_tl, o_it, o_lt,
           idx_v, nbr0, nbr1, nbr2):
        wid = lax.axis_index("s") * NC + lax.axis_index("c")

        def grab(idx_hbm, base, c, rels):
            pltpu.sync_copy(idx_hbm.at[pl.ds(base, c)], idx_v.at[pl.ds(0, c)])
            for hgt, nbr, o in rels:
                pltpu.sync_copy(hgt.at[idx_v.at[pl.ds(0, c)]],
                                nbr.at[pl.ds(0, c)])
                pltpu.sync_copy(nbr.at[pl.ds(0, c)], o.at[pl.ds(base, c)])

        rows_w = nb // NW

        def big(j, carry):
            base = wid * rows_w + j * IDXC
            grab(iids_h, base, IDXC,
                 [(h_ii, nbr0, o_ii), (h_ti, nbr1, o_ti), (h_li, nbr2, o_li)])
            return carry
        lax.fori_loop(0, rows_w // IDXC, big, 0)

        base_s = wid * (ns // NW)
        grab(locs_h, base_s, ns // NW,
             [(h_il, nbr0, o_il), (h_tl, nbr1, o_tl)])
        grab(times_h, base_s, ns // NW,
             [(h_it, nbr0, o_it), (h_lt, nbr1, o_lt)])

    return s1(iids, locs, times, hg['II'], hg['TI'], hg['LI'], hg['IL'],
              hg['TL'], hg['IT'], hg['LT'])


def _sc_gather_means(item_tab, loc_tab, time_tab, iids, locs, times,
                     f_ii, f_ti, f_li, f_il, f_tl, f_it, f_lt):
    nb = iids.shape[0]
    ns = locs.shape[0]
    rows_w = nb // NW
    n_chunks = rows_w // CHUNK
    srows_w = ns // NW
    f32 = jnp.float32

    out_type = tuple(
        jax.ShapeDtypeStruct(s, f32)
        for s in [(nb, D), (nb, D), (nb, D), (nb, D),
                  (ns, D), (ns, D), (ns, D),
                  (ns, D), (ns, D), (ns, D)]
    )
    scratch = [
        pltpu.VMEM((IDXC,), jnp.int32),
        pltpu.VMEM((IDXC, D), f32),
        pltpu.VMEM((FLAT,), jnp.int32),
        pltpu.VMEM((FLAT,), jnp.int32),
        pltpu.VMEM((FLAT, D), f32),
        pltpu.VMEM((FLAT, D), f32),
        pltpu.VMEM((CHUNK, D), f32),
        pltpu.SemaphoreType.DMA,
        pltpu.SemaphoreType.DMA,
    ]

    @functools.partial(
        pl.kernel, out_type=out_type, mesh=_mesh(), scratch_types=scratch,
        compiler_params=pltpu.CompilerParams(use_tc_tiling_on_sc=False))
    def s2(item_t, loc_t, time_t, iids_h, locs_h, times_h,
           fii, fti, fli, fil, ftl, fit, flt,
           o_self_i, o_mii, o_mti, o_mli,
           o_self_l, o_mil, o_mtl,
           o_self_t, o_mit, o_mlt,
           sidx, srows, fidx0, fidx1, rows0, rows1, mbuf, sem0, sem1):
        wid = lax.axis_index("s") * NC + lax.axis_index("c")
        fidx = (fidx0, fidx1)
        rows_b = (rows0, rows1)
        sems = (sem0, sem1)

        def reduce_mean(rows):
            def body(c, carry):
                r0 = c * K
                for d in range(D // 16):
                    sl = pl.ds(d * 16, 16)
                    acc = rows[r0, sl]
                    for kk in range(1, K):
                        acc = acc + rows[r0 + kk, sl]
                    mbuf[c, sl] = acc * (1.0 / K)
                return carry
            lax.fori_loop(0, CHUNK, body, 0)

        def self_job(idx_hbm, n, tab, out):
            per_w = n // NW

            def one(j, carry):
                c = min(IDXC, per_w)
                base = wid * per_w + j * c
                pltpu.sync_copy(idx_hbm.at[pl.ds(base, c)],
                                sidx.at[pl.ds(0, c)])
                pltpu.sync_copy(tab.at[sidx.at[pl.ds(0, c)]],
                                srows.at[pl.ds(0, c)])
                pltpu.sync_copy(srows.at[pl.ds(0, c)],
                                out.at[pl.ds(base, c)])
                return carry
            lax.fori_loop(0, max(1, per_w // IDXC), one, 0)

        def mean_job(flat_hbm, n, tab, out):
            per_w = n // NW
            nch = per_w // CHUNK

            def start(j, b):
                pltpu.sync_copy(
                    flat_hbm.at[pl.ds((wid * per_w + j * CHUNK) * K, FLAT)],
                    fidx[b])
                pltpu.async_copy(tab.at[fidx[b]], rows_b[b], sems[b])

            def finish(j, b):
                pltpu.make_async_copy(tab.at[fidx[b]], rows_b[b],
                                      sems[b]).wait()
                reduce_mean(rows_b[b])
                pltpu.sync_copy(mbuf,
                                out.at[pl.ds(wid * per_w + j * CHUNK, CHUNK)])

            if nch == 1:
                start(0, 0)
                finish(0, 0)
                return
            start(0, 0)
            start(1, 1)

            def outer(jj, carry):
                for b in (0, 1):
                    j = jj * 2 + b
                    finish(j, b)
                    nxt = j + 2

                    @pl.when(nxt < nch)
                    def _():
                        start(nxt, b)
                return carry
            lax.fori_loop(0, nch // 2, outer, 0)

        self_job(iids_h, nb, item_t, o_self_i)
        self_job(locs_h, ns, loc_t, o_self_l)
        self_job(times_h, ns, time_t, o_self_t)
        mean_job(fii, nb, item_t, o_mii)
        mean_job(fti, nb, time_t, o_mti)
        mean_job(fli, nb, loc_t, o_mli)
        mean_job(fil, ns, item_t, o_mil)
        mean_job(ftl, ns, time_t, o_mtl)
        mean_job(fit, ns, item_t, o_mit)
        mean_job(flt, ns, loc_t, o_mlt)

    return s2(item_tab, loc_tab, time_tab, iids, locs, times,
              f_ii, f_ti, f_li, f_il, f_tl, f_it, f_lt)


def _att_head(self_rows, aggs, ws, bs, sess, att_w1, att_b1, att_q):
    n, d = self_rows.shape
    n_agg = len(aggs)
    m = n_agg + (1 if sess is not None else 0)
    r = min(512, n)
    g = n // r
    f32 = jnp.float32
    inv_n = 1.0 / n

    row_spec = pl.BlockSpec((r, d), lambda i: (i, 0))
    w_spec = pl.BlockSpec((2 * d, d), lambda i: (0, 0))
    b_spec = pl.BlockSpec((1, d), lambda i: (0, 0))
    w1_spec = pl.BlockSpec((d, d), lambda i: (0, 0))
    q_spec = pl.BlockSpec((d, 1), lambda i: (0, 0))
    smem_spec = pl.BlockSpec(memory_space=pltpu.SMEM)

    sess_in = [sess] if sess is not None else []
    sess_specs = [row_spec] if sess is not None else []

    def heads(self_b, agg_b, sess_b, w_b, b_b):
        hs = []
        for p in range(n_agg):
            h = (jnp.dot(self_b, w_b[p][:d, :], preferred_element_type=f32)
                 + jnp.dot(agg_b[p], w_b[p][d:, :], preferred_element_type=f32)
                 + b_b[p])
            hs.append(jnp.maximum(h, 0.0))
        if sess_b is not None:
            hs.append(sess_b)
        return hs

    def body_a(*args):
        i = pl.program_id(0)
        it = iter(args)
        self_b = next(it)[...]
        agg_b = [next(it)[...] for _ in range(n_agg)]
        sess_b = next(it)[...] if sess is not None else None
        w_b = [next(it)[...] for _ in range(n_agg)]
        b_b = [next(it)[...] for _ in range(n_agg)]
        w1_b = next(it)[...]
        b1_b = next(it)[...]
        q_b = next(it)[...]
        wsum_ref = next(it)
        hs = heads(self_b, agg_b, sess_b, w_b, b_b)

        @pl.when(i == 0)
        def _():
            for p in range(m):
                wsum_ref[p] = 0.0

        for p in range(m):
            w = jnp.dot(
                jnp.tanh(jnp.dot(hs[p], w1_b, preferred_element_type=f32)
                         + b1_b),
                q_b, preferred_element_type=f32)
            wsum_ref[p] += jnp.sum(w)

    wsum = pl.pallas_call(
        body_a,
        grid=(g,),
        in_specs=[row_spec] + [row_spec] * n_agg + sess_specs
                 + [w_spec] * n_agg + [b_spec] * n_agg
                 + [w1_spec, b_spec, q_spec],
        out_specs=smem_spec,
        out_shape=jax.ShapeDtypeStruct((m,), f32),
    )(self_rows, *aggs, *sess_in, *ws, *bs, att_w1, att_b1, att_q)

    def body_b(*args):
        it = iter(args)
        self_b = next(it)[...]
        agg_b = [next(it)[...] for _ in range(n_agg)]
        sess_b = next(it)[...] if sess is not None else None
        w_b = [next(it)[...] for _ in range(n_agg)]
        b_b = [next(it)[...] for _ in range(n_agg)]
        wsum_ref = next(it)
        out_ref = next(it)
        hs = heads(self_b, agg_b, sess_b, w_b, b_b)

        sv = [jnp.full((1, 1), wsum_ref[p] * inv_n, f32) for p in range(m)]
        mx = sv[0]
        for p in range(1, m):
            mx = jnp.maximum(mx, sv[p])
        ev = [jnp.exp(sv[p] - mx) for p in range(m)]
        denom = ev[0]
        for p in range(1, m):
            denom = denom + ev[p]
        acc = hs[0] * ev[0]
        for p in range(1, m):
            acc = acc + hs[p] * ev[p]
        out_ref[...] = acc / denom

    return pl.pallas_call(
        body_b,
        grid=(g,),
        in_specs=[row_spec] + [row_spec] * n_agg + sess_specs
                 + [w_spec] * n_agg + [b_spec] * n_agg + [smem_spec],
        out_specs=row_spec,
        out_shape=jax.ShapeDtypeStruct((n, d), f32),
    )(self_rows, *aggs, *sess_in, *ws, *bs, wsum)


def kernel(item, locs, times, session_emb, params, Hg):
    iids = item.reshape(-1)
    hg_rows = _sc_gather_hg(iids, locs, times, Hg)
    flats = tuple(a.reshape(-1) for a in hg_rows)
    (self_i, mii, mti, mli, self_l, mil, mtl, self_t, mit, mlt) = \
        _sc_gather_means(params['item_tab'], params['loc_tab'],
                         params['time_tab'], iids, locs, times, *flats)
    sess = session_emb[:, 0, :]

    def b2(name):
        return params[name].reshape(1, D)

    h_items = _att_head(
        self_i, [mii, mti, mli],
        [params['II_W'], params['TI_W'], params['LI_W']],
        [b2('II_b'), b2('TI_b'), b2('LI_b')],
        None, params['att_i_W1'], b2('att_i_b1'), params['att_i_q'])
    h_locs = _att_head(
        self_l, [mil, mtl],
        [params['IL_W'], params['TL_W']],
        [b2('IL_b'), b2('TL_b')],
        sess, params['att_l_W1'], b2('att_l_b1'), params['att_l_q'])
    h_times = _att_head(
        self_t, [mit, mlt],
        [params['IT_W'], params['LT_W']],
        [b2('IT_b'), b2('LT_b')],
        sess, params['att_t_W1'], b2('att_t_b1'), params['att_t_q'])
    return (h_items, h_locs, h_times)


# speedup vs baseline: 8.5573x; 1.0186x over previous
import functools

import jax
import jax.numpy as jnp
from jax import lax
from jax.experimental import pallas as pl
from jax.experimental.pallas import tpu as pltpu
from jax.experimental.pallas import tpu_sc as plsc

D = 64
K = 16
CHUNK = 32
FLAT = CHUNK * K
SUB = 512
NC = 2
NS = 16
NW = NC * NS


IDXC = 80


def _mesh():
    return plsc.VectorSubcoreMesh(
        core_axis_name="c", subcore_axis_name="s", num_cores=NC,
        num_subcores=NS)


def _sc_gather_hg(iids, locs, times, hg):
    nb = iids.shape[0]
    ns = locs.shape[0]
    i32 = jnp.int32
    out_type = tuple(
        jax.ShapeDtypeStruct(s, i32)
        for s in [(nb, K), (nb, K), (nb, K),
                  (ns, K), (ns, K), (ns, K), (ns, K)]
    )
    scratch = [
        pltpu.VMEM((IDXC,), i32),
        pltpu.VMEM((IDXC, K), i32),
        pltpu.VMEM((IDXC, K), i32),
        pltpu.VMEM((IDXC, K), i32),
    ]

    @functools.partial(
        pl.kernel, out_type=out_type, mesh=_mesh(), scratch_types=scratch,
        compiler_params=pltpu.CompilerParams(use_tc_tiling_on_sc=False))
    def s1(iids_h, locs_h, times_h, h_ii, h_ti, h_li, h_il, h_tl, h_it, h_lt,
           o_ii, o_ti, o_li, o_il, o_tl, o_it, o_lt,
           idx_v, nbr0, nbr1, nbr2):
        wid = lax.axis_index("s") * NC + lax.axis_index("c")

        def grab(idx_hbm, base, c, rels):
            pltpu.sync_copy(idx_hbm.at[pl.ds(base, c)], idx_v.at[pl.ds(0, c)])
            for hgt, nbr, o in rels:
                pltpu.sync_copy(hgt.at[idx_v.at[pl.ds(0, c)]],
                                nbr.at[pl.ds(0, c)])
                pltpu.sync_copy(nbr.at[pl.ds(0, c)], o.at[pl.ds(base, c)])

        rows_w = nb // NW

        def big(j, carry):
            base = wid * rows_w + j * IDXC
            grab(iids_h, base, IDXC,
                 [(h_ii, nbr0, o_ii), (h_ti, nbr1, o_ti), (h_li, nbr2, o_li)])
            return carry
        lax.fori_loop(0, rows_w // IDXC, big, 0)

        base_s = wid * (ns // NW)
        grab(locs_h, base_s, ns // NW,
             [(h_il, nbr0, o_il), (h_tl, nbr1, o_tl)])
        grab(times_h, base_s, ns // NW,
             [(h_it, nbr0, o_it), (h_lt, nbr1, o_lt)])

    return s1(iids, locs, times, hg['II'], hg['TI'], hg['LI'], hg['IL'],
              hg['TL'], hg['IT'], hg['LT'])


def _sc_gather_means(item_tab, loc_tab, time_tab, iids, locs, times,
                     f_ii, f_ti, f_li, f_il, f_tl, f_it, f_lt):
    nb = iids.shape[0]
    ns = locs.shape[0]
    rows_w = nb // NW
    n_chunks = rows_w // CHUNK
    srows_w = ns // NW
    f32 = jnp.float32

    out_type = tuple(
        jax.ShapeDtypeStruct(s, f32)
        for s in [(nb, D), (nb, D), (nb, D), (nb, D),
                  (ns, D), (ns, D), (ns, D),
                  (ns, D), (ns, D), (ns, D)]
    )
    scratch = [
        pltpu.VMEM((IDXC,), jnp.int32),
        pltpu.VMEM((IDXC, D), f32),
        pltpu.VMEM((FLAT,), jnp.int32),
        pltpu.VMEM((FLAT,), jnp.int32),
        pltpu.VMEM((FLAT, D), f32),
        pltpu.VMEM((FLAT, D), f32),
        pltpu.VMEM((CHUNK, D), f32),
        pltpu.SemaphoreType.DMA,
        pltpu.SemaphoreType.DMA,
    ]

    @functools.partial(
        pl.kernel, out_type=out_type, mesh=_mesh(), scratch_types=scratch,
        compiler_params=pltpu.CompilerParams(use_tc_tiling_on_sc=False))
    def s2(item_t, loc_t, time_t, iids_h, locs_h, times_h,
           fii, fti, fli, fil, ftl, fit, flt,
           o_self_i, o_mii, o_mti, o_mli,
           o_self_l, o_mil, o_mtl,
           o_self_t, o_mit, o_mlt,
           sidx, srows, fidx0, fidx1, rows0, rows1, mbuf, sem0, sem1):
        wid = lax.axis_index("s") * NC + lax.axis_index("c")
        fidx = (fidx0, fidx1)
        rows_b = (rows0, rows1)
        sems = (sem0, sem1)

        def reduce_mean(rows):
            def body(c, carry):
                r0 = c * K
                for d in range(D // 16):
                    sl = pl.ds(d * 16, 16)
                    acc = rows[r0, sl]
                    for kk in range(1, K):
                        acc = acc + rows[r0 + kk, sl]
                    mbuf[c, sl] = acc * (1.0 / K)
                return carry
            lax.fori_loop(0, CHUNK, body, 0)

        def self_job(idx_hbm, n, tab, out):
            per_w = n // NW

            def one(j, carry):
                c = min(IDXC, per_w)
                base = wid * per_w + j * c
                pltpu.sync_copy(idx_hbm.at[pl.ds(base, c)],
                                sidx.at[pl.ds(0, c)])
                pltpu.sync_copy(tab.at[sidx.at[pl.ds(0, c)]],
                                srows.at[pl.ds(0, c)])
                pltpu.sync_copy(srows.at[pl.ds(0, c)],
                                out.at[pl.ds(base, c)])
                return carry
            lax.fori_loop(0, max(1, per_w // IDXC), one, 0)

        def mean_job(flat_hbm, n, tab, out):
            per_w = n // NW
            nch = per_w // CHUNK

            def start(j, b):
                pltpu.sync_copy(
                    flat_hbm.at[pl.ds((wid * per_w + j * CHUNK) * K, FLAT)],
                    fidx[b])
                pltpu.async_copy(tab.at[fidx[b]], rows_b[b], sems[b])

            def finish(j, b):
                pltpu.make_async_copy(tab.at[fidx[b]], rows_b[b],
                                      sems[b]).wait()
                reduce_mean(rows_b[b])
                pltpu.sync_copy(mbuf,
                                out.at[pl.ds(wid * per_w + j * CHUNK, CHUNK)])

            if nch == 1:
                start(0, 0)
                finish(0, 0)
                return
            start(0, 0)
            start(1, 1)

            def outer(jj, carry):
                for b in (0, 1):
                    j = jj * 2 + b
                    finish(j, b)
                    nxt = j + 2

                    @pl.when(nxt < nch)
                    def _():
                        start(nxt, b)
                return carry
            lax.fori_loop(0, nch // 2, outer, 0)

        self_job(iids_h, nb, item_t, o_self_i)
        self_job(locs_h, ns, loc_t, o_self_l)
        self_job(times_h, ns, time_t, o_self_t)
        mean_job(fii, nb, item_t, o_mii)
        mean_job(fti, nb, time_t, o_mti)
        mean_job(fli, nb, loc_t, o_mli)
        mean_job(fil, ns, item_t, o_mil)
        mean_job(ftl, ns, time_t, o_mtl)
        mean_job(fit, ns, item_t, o_mit)
        mean_job(flt, ns, loc_t, o_mlt)

    return s2(item_tab, loc_tab, time_tab, iids, locs, times,
              f_ii, f_ti, f_li, f_il, f_tl, f_it, f_lt)


def _att_head(self_rows, aggs, ws, bs, sess, att_w1, att_b1, att_q):
    n, d = self_rows.shape
    n_agg = len(aggs)
    m = n_agg + (1 if sess is not None else 0)
    kx = (1 + n_agg) * d
    hw = n_agg * d
    zw = m * d
    r = min(512, n)
    g = n // r
    f32 = jnp.float32
    inv_n = 1.0 / n

    wcat = jnp.zeros((kx, hw), f32)
    for p in range(n_agg):
        wcat = wcat.at[0:d, p * d:(p + 1) * d].set(ws[p][:d, :])
        wcat = wcat.at[(1 + p) * d:(2 + p) * d, p * d:(p + 1) * d].set(
            ws[p][d:, :])
    bcat = jnp.concatenate(bs, axis=1)
    w1d = jnp.kron(jnp.eye(m, dtype=f32), att_w1)
    b1d = jnp.tile(att_b1, (1, m))
    qd = jnp.kron(jnp.eye(m, dtype=f32), att_q)

    row64 = pl.BlockSpec((r, d), lambda i: (i, 0))
    rowh = pl.BlockSpec((r, hw), lambda i: (i, 0))
    smem_spec = pl.BlockSpec(memory_space=pltpu.SMEM)

    def full(a):
        return pl.BlockSpec(a.shape, lambda i: tuple(0 for _ in a.shape))

    sess_in = [sess] if sess is not None else []
    sess_specs = [row64] if sess is not None else []

    def body_a(*args):
        i = pl.program_id(0)
        it = iter(args)
        self_b = next(it)[...]
        agg_b = [next(it)[...] for _ in range(n_agg)]
        sess_b = next(it)[...] if sess is not None else None
        wcat_b = next(it)[...]
        bcat_b = next(it)[...]
        w1d_b = next(it)[...]
        b1d_b = next(it)[...]
        qd_b = next(it)[...]
        hcat_ref = next(it)
        wsum_ref = next(it)

        x = jnp.concatenate([self_b] + agg_b, axis=1)
        h = jnp.maximum(
            jnp.dot(x, wcat_b, preferred_element_type=f32) + bcat_b, 0.0)
        hcat_ref[...] = h
        z = h if sess_b is None else jnp.concatenate([h, sess_b], axis=1)
        t = jnp.tanh(jnp.dot(z, w1d_b, preferred_element_type=f32) + b1d_b)
        w3 = jnp.dot(t, qd_b, preferred_element_type=f32)

        @pl.when(i == 0)
        def _():
            for p in range(m):
                wsum_ref[p] = 0.0

        for p in range(m):
            wsum_ref[p] += jnp.sum(w3[:, p:p + 1])

    hcat, wsum = pl.pallas_call(
        body_a,
        grid=(g,),
        in_specs=[row64] + [row64] * n_agg + sess_specs
                 + [full(wcat), full(bcat), full(w1d), full(b1d), full(qd)],
        out_specs=(rowh, smem_spec),
        out_shape=(jax.ShapeDtypeStruct((n, hw), f32),
                   jax.ShapeDtypeStruct((m,), f32)),
    )(self_rows, *aggs, *sess_in, wcat, bcat, w1d, b1d, qd)

    def body_b(*args):
        it = iter(args)
        hcat_b = next(it)[...]
        sess_b = next(it)[...] if sess is not None else None
        wsum_ref = next(it)
        out_ref = next(it)

        parts = [hcat_b[:, p * d:(p + 1) * d] for p in range(n_agg)]
        if sess_b is not None:
            parts.append(sess_b)
        sv = [jnp.full((1, 1), wsum_ref[p] * inv_n, f32) for p in range(m)]
        mx = sv[0]
        for p in range(1, m):
            mx = jnp.maximum(mx, sv[p])
        ev = [jnp.exp(sv[p] - mx) for p in range(m)]
        denom = ev[0]
        for p in range(1, m):
            denom = denom + ev[p]
        acc = parts[0] * ev[0]
        for p in range(1, m):
            acc = acc + parts[p] * ev[p]
        out_ref[...] = acc / denom

    return pl.pallas_call(
        body_b,
        grid=(g,),
        in_specs=[rowh] + sess_specs + [smem_spec],
        out_specs=row64,
        out_shape=jax.ShapeDtypeStruct((n, d), f32),
    )(hcat, *sess_in, wsum)


def kernel(item, locs, times, session_emb, params, Hg):
    iids = item.reshape(-1)
    hg_rows = _sc_gather_hg(iids, locs, times, Hg)
    flats = tuple(a.reshape(-1) for a in hg_rows)
    (self_i, mii, mti, mli, self_l, mil, mtl, self_t, mit, mlt) = \
        _sc_gather_means(params['item_tab'], params['loc_tab'],
                         params['time_tab'], iids, locs, times, *flats)
    sess = session_emb[:, 0, :]

    def b2(name):
        return params[name].reshape(1, D)

    h_items = _att_head(
        self_i, [mii, mti, mli],
        [params['II_W'], params['TI_W'], params['LI_W']],
        [b2('II_b'), b2('TI_b'), b2('LI_b')],
        None, params['att_i_W1'], b2('att_i_b1'), params['att_i_q'])
    h_locs = _att_head(
        self_l, [mil, mtl],
        [params['IL_W'], params['TL_W']],
        [b2('IL_b'), b2('TL_b')],
        sess, params['att_l_W1'], b2('att_l_b1'), params['att_l_q'])
    h_times = _att_head(
        self_t, [mit, mlt],
        [params['IT_W'], params['LT_W']],
        [b2('IT_b'), b2('LT_b')],
        sess, params['att_t_W1'], b2('att_t_b1'), params['att_t_q'])
    return (h_items, h_locs, h_times)


# speedup vs baseline: 8.7741x; 1.0253x over previous
import functools

import jax
import jax.numpy as jnp
from jax import lax
from jax.experimental import pallas as pl
from jax.experimental.pallas import tpu as pltpu
from jax.experimental.pallas import tpu_sc as plsc

D = 64
K = 16
CHUNK = 32
FLAT = CHUNK * K
SUB = 512
NC = 2
NS = 16
NW = NC * NS


IDXC = 80


def _mesh():
    return plsc.VectorSubcoreMesh(
        core_axis_name="c", subcore_axis_name="s", num_cores=NC,
        num_subcores=NS)


def _sc_gather_hg(iids, locs, times, hg):
    nb = iids.shape[0]
    ns = locs.shape[0]
    i32 = jnp.int32
    out_type = tuple(
        jax.ShapeDtypeStruct(s, i32)
        for s in [(nb, K), (nb, K), (nb, K),
                  (ns, K), (ns, K), (ns, K), (ns, K)]
    )
    scratch = [
        pltpu.VMEM((IDXC,), i32),
        pltpu.VMEM((IDXC, K), i32),
        pltpu.VMEM((IDXC, K), i32),
        pltpu.VMEM((IDXC, K), i32),
    ]

    @functools.partial(
        pl.kernel, out_type=out_type, mesh=_mesh(), scratch_types=scratch,
        compiler_params=pltpu.CompilerParams(use_tc_tiling_on_sc=False))
    def s1(iids_h, locs_h, times_h, h_ii, h_ti, h_li, h_il, h_tl, h_it, h_lt,
           o_ii, o_ti, o_li, o_il, o_tl, o_it, o_lt,
           idx_v, nbr0, nbr1, nbr2):
        wid = lax.axis_index("s") * NC + lax.axis_index("c")

        def grab(idx_hbm, base, c, rels):
            pltpu.sync_copy(idx_hbm.at[pl.ds(base, c)], idx_v.at[pl.ds(0, c)])
            for hgt, nbr, o in rels:
                pltpu.sync_copy(hgt.at[idx_v.at[pl.ds(0, c)]],
                                nbr.at[pl.ds(0, c)])
                pltpu.sync_copy(nbr.at[pl.ds(0, c)], o.at[pl.ds(base, c)])

        rows_w = nb // NW

        def big(j, carry):
            base = wid * rows_w + j * IDXC
            grab(iids_h, base, IDXC,
                 [(h_ii, nbr0, o_ii), (h_ti, nbr1, o_ti), (h_li, nbr2, o_li)])
            return carry
        lax.fori_loop(0, rows_w // IDXC, big, 0)

        base_s = wid * (ns // NW)
        grab(locs_h, base_s, ns // NW,
             [(h_il, nbr0, o_il), (h_tl, nbr1, o_tl)])
        grab(times_h, base_s, ns // NW,
             [(h_it, nbr0, o_it), (h_lt, nbr1, o_lt)])

    return s1(iids, locs, times, hg['II'], hg['TI'], hg['LI'], hg['IL'],
              hg['TL'], hg['IT'], hg['LT'])


def _sc_gather_means(item_tab, loc_tab, time_tab, iids, locs, times,
                     f_ii, f_ti, f_li, f_il, f_tl, f_it, f_lt):
    nb = iids.shape[0]
    ns = locs.shape[0]
    rows_w = nb // NW
    n_chunks = rows_w // CHUNK
    srows_w = ns // NW
    f32 = jnp.float32

    out_type = tuple(
        jax.ShapeDtypeStruct(s, f32)
        for s in [(nb, D), (nb, D), (nb, D), (nb, D),
                  (ns, D), (ns, D), (ns, D),
                  (ns, D), (ns, D), (ns, D)]
    )
    scratch = [
        pltpu.VMEM((IDXC,), jnp.int32),
        pltpu.VMEM((IDXC,), jnp.int32),
        pltpu.VMEM((IDXC, D), f32),
        pltpu.VMEM((IDXC, D), f32),
        pltpu.VMEM((FLAT,), jnp.int32),
        pltpu.VMEM((FLAT,), jnp.int32),
        pltpu.VMEM((FLAT, D), f32),
        pltpu.VMEM((FLAT, D), f32),
        pltpu.VMEM((CHUNK, D), f32),
        pltpu.VMEM((CHUNK, D), f32),
        pltpu.SemaphoreType.DMA,
        pltpu.SemaphoreType.DMA,
        pltpu.SemaphoreType.DMA,
        pltpu.SemaphoreType.DMA,
    ]

    @functools.partial(
        pl.kernel, out_type=out_type, mesh=_mesh(), scratch_types=scratch,
        compiler_params=pltpu.CompilerParams(use_tc_tiling_on_sc=False))
    def s2(item_t, loc_t, time_t, iids_h, locs_h, times_h,
           fii, fti, fli, fil, ftl, fit, flt,
           o_self_i, o_mii, o_mti, o_mli,
           o_self_l, o_mil, o_mtl,
           o_self_t, o_mit, o_mlt,
           sidx0, sidx1, srows0, srows1, fidx0, fidx1, rows0, rows1,
           mbuf0, mbuf1, sem0, sem1, osem0, osem1):
        wid = lax.axis_index("s") * NC + lax.axis_index("c")
        sidx = (sidx0, sidx1)
        srows = (srows0, srows1)
        fidx = (fidx0, fidx1)
        rows_b = (rows0, rows1)
        mbuf = (mbuf0, mbuf1)
        sems = (sem0, sem1)
        osems = (osem0, osem1)

        def reduce_mean(rows, mb):
            def body(c, carry):
                r0 = c * K
                for d in range(D // 16):
                    sl = pl.ds(d * 16, 16)
                    acc = rows[r0, sl]
                    for kk in range(1, K):
                        acc = acc + rows[r0 + kk, sl]
                    mb[c, sl] = acc * (1.0 / K)
                return carry
            lax.fori_loop(0, CHUNK, body, 0)

        def self_job(idx_hbm, n, tab, out):
            per_w = n // NW
            c = min(IDXC, per_w)
            nch = per_w // c

            def start(j, b):
                base = wid * per_w + j * c
                pltpu.sync_copy(idx_hbm.at[pl.ds(base, c)],
                                sidx[b].at[pl.ds(0, c)])
                pltpu.async_copy(tab.at[sidx[b].at[pl.ds(0, c)]],
                                 srows[b].at[pl.ds(0, c)], sems[b])

            def finish(j, b):
                base = wid * per_w + j * c
                pltpu.make_async_copy(tab.at[sidx[b].at[pl.ds(0, c)]],
                                      srows[b].at[pl.ds(0, c)],
                                      sems[b]).wait()
                pltpu.sync_copy(srows[b].at[pl.ds(0, c)],
                                out.at[pl.ds(base, c)])

            if nch == 1:
                start(0, 0)
                finish(0, 0)
                return
            start(0, 0)
            start(1, 1)

            def outer(jj, carry):
                for b in (0, 1):
                    j = jj * 2 + b
                    finish(j, b)
                    nxt = j + 2

                    @pl.when(nxt < nch)
                    def _():
                        start(nxt, b)
                return carry
            lax.fori_loop(0, nch // 2, outer, 0)

        def mean_job(flat_hbm, n, tab, out):
            per_w = n // NW
            nch = per_w // CHUNK

            def start(j, b):
                pltpu.sync_copy(
                    flat_hbm.at[pl.ds((wid * per_w + j * CHUNK) * K, FLAT)],
                    fidx[b])
                pltpu.async_copy(tab.at[fidx[b]], rows_b[b], sems[b])

            def store_wait(j, b):
                pltpu.make_async_copy(
                    mbuf[b], out.at[pl.ds(wid * per_w + j * CHUNK, CHUNK)],
                    osems[b]).wait()

            def finish(j, b, drain):
                pltpu.make_async_copy(tab.at[fidx[b]], rows_b[b],
                                      sems[b]).wait()

                @pl.when(drain)
                def _():
                    store_wait(j - 2, b)
                reduce_mean(rows_b[b], mbuf[b])
                pltpu.async_copy(
                    mbuf[b], out.at[pl.ds(wid * per_w + j * CHUNK, CHUNK)],
                    osems[b])

            if nch == 1:
                start(0, 0)
                finish(0, 0, jnp.bool_(False))
                store_wait(0, 0)
                return
            start(0, 0)
            start(1, 1)

            def outer(jj, carry):
                for b in (0, 1):
                    j = jj * 2 + b
                    finish(j, b, j >= 2)
                    nxt = j + 2

                    @pl.when(nxt < nch)
                    def _():
                        start(nxt, b)
                return carry
            lax.fori_loop(0, nch // 2, outer, 0)
            store_wait(nch - 2, 0)
            store_wait(nch - 1, 1)

        self_job(iids_h, nb, item_t, o_self_i)
        self_job(locs_h, ns, loc_t, o_self_l)
        self_job(times_h, ns, time_t, o_self_t)
        mean_job(fii, nb, item_t, o_mii)
        mean_job(fti, nb, time_t, o_mti)
        mean_job(fli, nb, loc_t, o_mli)
        mean_job(fil, ns, item_t, o_mil)
        mean_job(ftl, ns, time_t, o_mtl)
        mean_job(fit, ns, item_t, o_mit)
        mean_job(flt, ns, loc_t, o_mlt)

    return s2(item_tab, loc_tab, time_tab, iids, locs, times,
              f_ii, f_ti, f_li, f_il, f_tl, f_it, f_lt)


def _att_head(self_rows, aggs, ws, bs, sess, att_w1, att_b1, att_q):
    n, d = self_rows.shape
    n_agg = len(aggs)
    m = n_agg + (1 if sess is not None else 0)
    kx = (1 + n_agg) * d
    hw = n_agg * d
    zw = m * d
    r = min(512, n)
    g = n // r
    f32 = jnp.float32
    inv_n = 1.0 / n

    wcat = jnp.zeros((kx, hw), f32)
    for p in range(n_agg):
        wcat = wcat.at[0:d, p * d:(p + 1) * d].set(ws[p][:d, :])
        wcat = wcat.at[(1 + p) * d:(2 + p) * d, p * d:(p + 1) * d].set(
            ws[p][d:, :])
    bcat = jnp.concatenate(bs, axis=1)
    w1d = jnp.kron(jnp.eye(m, dtype=f32), att_w1)
    b1d = jnp.tile(att_b1, (1, m))
    qd = jnp.kron(jnp.eye(m, dtype=f32), att_q)

    row64 = pl.BlockSpec((r, d), lambda i: (i, 0))
    rowh = pl.BlockSpec((r, hw), lambda i: (i, 0))
    smem_spec = pl.BlockSpec(memory_space=pltpu.SMEM)

    def full(a):
        return pl.BlockSpec(a.shape, lambda i: tuple(0 for _ in a.shape))

    sess_in = [sess] if sess is not None else []
    sess_specs = [row64] if sess is not None else []

    def body_a(*args):
        i = pl.program_id(0)
        it = iter(args)
        self_b = next(it)[...]
        agg_b = [next(it)[...] for _ in range(n_agg)]
        sess_b = next(it)[...] if sess is not None else None
        wcat_b = next(it)[...]
        bcat_b = next(it)[...]
        w1d_b = next(it)[...]
        b1d_b = next(it)[...]
        qd_b = next(it)[...]
        hcat_ref = next(it)
        wsum_ref = next(it)

        x = jnp.concatenate([self_b] + agg_b, axis=1)
        h = jnp.maximum(
            jnp.dot(x, wcat_b, preferred_element_type=f32) + bcat_b, 0.0)
        hcat_ref[...] = h
        z = h if sess_b is None else jnp.concatenate([h, sess_b], axis=1)
        t = jnp.tanh(jnp.dot(z, w1d_b, preferred_element_type=f32) + b1d_b)
        w3 = jnp.dot(t, qd_b, preferred_element_type=f32)

        @pl.when(i == 0)
        def _():
            for p in range(m):
                wsum_ref[p] = 0.0

        for p in range(m):
            wsum_ref[p] += jnp.sum(w3[:, p:p + 1])

    hcat, wsum = pl.pallas_call(
        body_a,
        grid=(g,),
        in_specs=[row64] + [row64] * n_agg + sess_specs
                 + [full(wcat), full(bcat), full(w1d), full(b1d), full(qd)],
        out_specs=(rowh, smem_spec),
        out_shape=(jax.ShapeDtypeStruct((n, hw), f32),
                   jax.ShapeDtypeStruct((m,), f32)),
    )(self_rows, *aggs, *sess_in, wcat, bcat, w1d, b1d, qd)

    def body_b(*args):
        it = iter(args)
        hcat_b = next(it)[...]
        sess_b = next(it)[...] if sess is not None else None
        wsum_ref = next(it)
        out_ref = next(it)

        parts = [hcat_b[:, p * d:(p + 1) * d] for p in range(n_agg)]
        if sess_b is not None:
            parts.append(sess_b)
        sv = [jnp.full((1, 1), wsum_ref[p] * inv_n, f32) for p in range(m)]
        mx = sv[0]
        for p in range(1, m):
            mx = jnp.maximum(mx, sv[p])
        ev = [jnp.exp(sv[p] - mx) for p in range(m)]
        denom = ev[0]
        for p in range(1, m):
            denom = denom + ev[p]
        acc = parts[0] * ev[0]
        for p in range(1, m):
            acc = acc + parts[p] * ev[p]
        out_ref[...] = acc / denom

    return pl.pallas_call(
        body_b,
        grid=(g,),
        in_specs=[rowh] + sess_specs + [smem_spec],
        out_specs=row64,
        out_shape=jax.ShapeDtypeStruct((n, d), f32),
    )(hcat, *sess_in, wsum)


def kernel(item, locs, times, session_emb, params, Hg):
    iids = item.reshape(-1)
    hg_rows = _sc_gather_hg(iids, locs, times, Hg)
    flats = tuple(a.reshape(-1) for a in hg_rows)
    (self_i, mii, mti, mli, self_l, mil, mtl, self_t, mit, mlt) = \
        _sc_gather_means(params['item_tab'], params['loc_tab'],
                         params['time_tab'], iids, locs, times, *flats)
    sess = session_emb[:, 0, :]

    def b2(name):
        return params[name].reshape(1, D)

    h_items = _att_head(
        self_i, [mii, mti, mli],
        [params['II_W'], params['TI_W'], params['LI_W']],
        [b2('II_b'), b2('TI_b'), b2('LI_b')],
        None, params['att_i_W1'], b2('att_i_b1'), params['att_i_q'])
    h_locs = _att_head(
        self_l, [mil, mtl],
        [params['IL_W'], params['TL_W']],
        [b2('IL_b'), b2('TL_b')],
        sess, params['att_l_W1'], b2('att_l_b1'), params['att_l_q'])
    h_times = _att_head(
        self_t, [mit, mlt],
        [params['IT_W'], params['LT_W']],
        [b2('IT_b'), b2('LT_b')],
        sess, params['att_t_W1'], b2('att_t_b1'), params['att_t_q'])
    return (h_items, h_locs, h_times)


# speedup vs baseline: 11.2266x; 1.2795x over previous
import functools

import jax
import jax.numpy as jnp
from jax import lax
from jax.experimental import pallas as pl
from jax.experimental.pallas import tpu as pltpu
from jax.experimental.pallas import tpu_sc as plsc

D = 64
K = 16
CHUNK = 32
FLAT = CHUNK * K
SUB = 512
NC = 2
NS = 16
NW = NC * NS


IDXC = 80


def _mesh():
    return plsc.VectorSubcoreMesh(
        core_axis_name="c", subcore_axis_name="s", num_cores=NC,
        num_subcores=NS)


def _sc_all(item_tab, loc_tab, time_tab, iids, locs, times,
            g_ii, g_ti, g_li, g_il, g_tl, g_it, g_lt):
    nb = iids.shape[0]
    ns = locs.shape[0]
    rows_w = nb // NW
    f32 = jnp.float32
    i32 = jnp.int32

    out_type = tuple(
        jax.ShapeDtypeStruct(s, f32)
        for s in [(nb, D), (nb, D), (nb, D), (nb, D),
                  (ns, D), (ns, D), (ns, D),
                  (ns, D), (ns, D), (ns, D)]
    )
    scratch = [
        pltpu.VMEM((rows_w,), i32),
        pltpu.VMEM((CHUNK,), i32),
        pltpu.VMEM((CHUNK,), i32),
        pltpu.VMEM((IDXC, D), f32),
        pltpu.VMEM((IDXC, D), f32),
        pltpu.VMEM((FLAT,), i32),
        pltpu.VMEM((FLAT,), i32),
        pltpu.VMEM((FLAT,), i32),
        pltpu.VMEM((FLAT,), i32),
        pltpu.VMEM((FLAT, D), f32),
        pltpu.VMEM((FLAT, D), f32),
        pltpu.VMEM((CHUNK, D), f32),
        pltpu.VMEM((CHUNK, D), f32),
        pltpu.SemaphoreType.DMA,
        pltpu.SemaphoreType.DMA,
        pltpu.SemaphoreType.DMA,
        pltpu.SemaphoreType.DMA,
        pltpu.SemaphoreType.DMA,
        pltpu.SemaphoreType.DMA,
    ]

    @functools.partial(
        pl.kernel, out_type=out_type, mesh=_mesh(), scratch_types=scratch,
        compiler_params=pltpu.CompilerParams(use_tc_tiling_on_sc=False))
    def sc(item_t, loc_t, time_t, iids_h, locs_h, times_h,
           hii, hti, hli, hil, htl, hit, hlt,
           o_self_i, o_mii, o_mti, o_mli,
           o_self_l, o_mil, o_mtl,
           o_self_t, o_mit, o_mlt,
           aidx, lidx, tidx, srows0, srows1, fidx0, fidx1, nbr0, nbr1,
           rows0, rows1, mbuf0, mbuf1,
           nsem0, nsem1, esem0, esem1, osem0, osem1):
        wid = lax.axis_index("s") * NC + lax.axis_index("c")
        srows = (srows0, srows1)
        fidx = (fidx0, fidx1)
        nbr = (nbr0, nbr1)
        rows_b = (rows0, rows1)
        mbuf = (mbuf0, mbuf1)
        nsems = (nsem0, nsem1)
        esems = (esem0, esem1)
        osems = (osem0, osem1)

        pltpu.sync_copy(iids_h.at[pl.ds(wid * rows_w, rows_w)], aidx)
        pltpu.sync_copy(locs_h.at[pl.ds(wid * CHUNK, CHUNK)], lidx)
        pltpu.sync_copy(times_h.at[pl.ds(wid * CHUNK, CHUNK)], tidx)

        def reduce_mean(rows, mb):
            def body(c, carry):
                for d in range(D // 16):
                    sl = pl.ds(d * 16, 16)
                    acc = rows[c, sl]
                    for kk in range(1, K):
                        acc = acc + rows[kk * CHUNK + c, sl]
                    mb[c, sl] = acc * (1.0 / K)
                return carry
            lax.fori_loop(0, CHUNK, body, 0)

        def self_job(idx_all, n, tab, out):
            per_w = n // NW
            c = min(IDXC, per_w)
            nch = per_w // c

            def start(j, b):
                pltpu.async_copy(tab.at[idx_all.at[pl.ds(j * c, c)]],
                                 srows[b].at[pl.ds(0, c)], esems[b])

            def finish(j, b):
                pltpu.make_async_copy(tab.at[idx_all.at[pl.ds(j * c, c)]],
                                      srows[b].at[pl.ds(0, c)],
                                      esems[b]).wait()
                pltpu.sync_copy(srows[b].at[pl.ds(0, c)],
                                out.at[pl.ds(wid * per_w + j * c, c)])

            if nch == 1:
                start(0, 0)
                finish(0, 0)
                return
            start(0, 0)
            start(1, 1)

            def outer(jj, carry):
                for b in (0, 1):
                    j = jj * 2 + b
                    finish(j, b)
                    nxt = j + 2

                    @pl.when(nxt < nch)
                    def _():
                        start(nxt, b)
                return carry
            lax.fori_loop(0, nch // 2, outer, 0)

        def mean_job(hgf, mult, idx_all, n, tab, out):
            per_w = n // NW
            nch = per_w // CHUNK

            def start(j, b):
                fx = fidx[b]
                v0 = idx_all[pl.ds(j * CHUNK, 16)]
                v1 = idx_all[pl.ds(j * CHUNK + 16, 16)]
                for kk in range(K):
                    fx[pl.ds(kk * CHUNK, 16)] = v0 + kk * mult
                    fx[pl.ds(kk * CHUNK + 16, 16)] = v1 + kk * mult
                pltpu.async_copy(hgf.at[fx], nbr[b], nsems[b])

            def mid(j, b):
                pltpu.make_async_copy(hgf.at[fidx[b]], nbr[b],
                                      nsems[b]).wait()
                pltpu.async_copy(tab.at[nbr[b]], rows_b[b], esems[b])

            def store_wait(j, b):
                pltpu.make_async_copy(
                    mbuf[b], out.at[pl.ds(wid * per_w + j * CHUNK, CHUNK)],
                    osems[b]).wait()

            def finish(j, b, drain):
                pltpu.make_async_copy(tab.at[nbr[b]], rows_b[b],
                                      esems[b]).wait()

                @pl.when(drain)
                def _():
                    store_wait(j - 2, b)
                reduce_mean(rows_b[b], mbuf[b])
                pltpu.async_copy(
                    mbuf[b], out.at[pl.ds(wid * per_w + j * CHUNK, CHUNK)],
                    osems[b])

            if nch == 1:
                start(0, 0)
                mid(0, 0)
                finish(0, 0, jnp.bool_(False))
                store_wait(0, 0)
                return
            start(0, 0)
            start(1, 1)
            mid(0, 0)

            def outer(jj, carry):
                for b in (0, 1):
                    j = jj * 2 + b
                    pltpu.make_async_copy(tab.at[nbr[b]], rows_b[b],
                                          esems[b]).wait()

                    @pl.when(j >= 2)
                    def _():
                        store_wait(j - 2, b)

                    @pl.when(j + 2 < nch)
                    def _():
                        start(j + 2, b)

                    @pl.when(j + 1 < nch)
                    def _():
                        mid(j + 1, 1 - b)
                    reduce_mean(rows_b[b], mbuf[b])
                    pltpu.async_copy(
                        mbuf[b],
                        out.at[pl.ds(wid * per_w + j * CHUNK, CHUNK)],
                        osems[b])
                return carry
            lax.fori_loop(0, nch // 2, outer, 0)
            store_wait(nch - 2, 0)
            store_wait(nch - 1, 1)

        NI, NL, NT = item_t.shape[0], loc_t.shape[0], time_t.shape[0]
        mean_job(hii, NI, aidx, nb, item_t, o_mii)
        mean_job(hti, NI, aidx, nb, time_t, o_mti)
        mean_job(hli, NI, aidx, nb, loc_t, o_mli)
        self_job(aidx, nb, item_t, o_self_i)
        self_job(lidx, ns, loc_t, o_self_l)
        self_job(tidx, ns, time_t, o_self_t)
        mean_job(hil, NL, lidx, ns, item_t, o_mil)
        mean_job(htl, NL, lidx, ns, time_t, o_mtl)
        mean_job(hit, NT, tidx, ns, item_t, o_mit)
        mean_job(hlt, NT, tidx, ns, loc_t, o_mlt)

    return sc(item_tab, loc_tab, time_tab, iids, locs, times,
              g_ii, g_ti, g_li, g_il, g_tl, g_it, g_lt)


def _att_head(self_rows, aggs, ws, bs, sess, att_w1, att_b1, att_q):
    n, d = self_rows.shape
    n_agg = len(aggs)
    m = n_agg + (1 if sess is not None else 0)
    kx = (1 + n_agg) * d
    hw = n_agg * d
    zw = m * d
    r = min(512, n)
    g = n // r
    f32 = jnp.float32
    inv_n = 1.0 / n

    wcat = jnp.zeros((kx, hw), f32)
    for p in range(n_agg):
        wcat = wcat.at[0:d, p * d:(p + 1) * d].set(ws[p][:d, :])
        wcat = wcat.at[(1 + p) * d:(2 + p) * d, p * d:(p + 1) * d].set(
            ws[p][d:, :])
    bcat = jnp.concatenate(bs, axis=1)
    w1d = jnp.kron(jnp.eye(m, dtype=f32), att_w1)
    b1d = jnp.tile(att_b1, (1, m))
    qd = jnp.kron(jnp.eye(m, dtype=f32), att_q)

    row64 = pl.BlockSpec((r, d), lambda i: (i, 0))
    rowh = pl.BlockSpec((r, hw), lambda i: (i, 0))
    smem_spec = pl.BlockSpec(memory_space=pltpu.SMEM)

    def full(a):
        return pl.BlockSpec(a.shape, lambda i: tuple(0 for _ in a.shape))

    sess_in = [sess] if sess is not None else []
    sess_specs = [row64] if sess is not None else []

    def body_a(*args):
        i = pl.program_id(0)
        it = iter(args)
        self_b = next(it)[...]
        agg_b = [next(it)[...] for _ in range(n_agg)]
        sess_b = next(it)[...] if sess is not None else None
        wcat_b = next(it)[...]
        bcat_b = next(it)[...]
        w1d_b = next(it)[...]
        b1d_b = next(it)[...]
        qd_b = next(it)[...]
        hcat_ref = next(it)
        wsum_ref = next(it)

        x = jnp.concatenate([self_b] + agg_b, axis=1)
        h = jnp.maximum(
            jnp.dot(x, wcat_b, preferred_element_type=f32) + bcat_b, 0.0)
        hcat_ref[...] = h
        z = h if sess_b is None else jnp.concatenate([h, sess_b], axis=1)
        t = jnp.tanh(jnp.dot(z, w1d_b, preferred_element_type=f32) + b1d_b)
        w3 = jnp.dot(t, qd_b, preferred_element_type=f32)

        @pl.when(i == 0)
        def _():
            for p in range(m):
                wsum_ref[p] = 0.0

        for p in range(m):
            wsum_ref[p] += jnp.sum(w3[:, p:p + 1])

    hcat, wsum = pl.pallas_call(
        body_a,
        grid=(g,),
        in_specs=[row64] + [row64] * n_agg + sess_specs
                 + [full(wcat), full(bcat), full(w1d), full(b1d), full(qd)],
        out_specs=(rowh, smem_spec),
        out_shape=(jax.ShapeDtypeStruct((n, hw), f32),
                   jax.ShapeDtypeStruct((m,), f32)),
    )(self_rows, *aggs, *sess_in, wcat, bcat, w1d, b1d, qd)

    def body_b(*args):
        it = iter(args)
        hcat_b = next(it)[...]
        sess_b = next(it)[...] if sess is not None else None
        wsum_ref = next(it)
        out_ref = next(it)

        parts = [hcat_b[:, p * d:(p + 1) * d] for p in range(n_agg)]
        if sess_b is not None:
            parts.append(sess_b)
        sv = [jnp.full((1, 1), wsum_ref[p] * inv_n, f32) for p in range(m)]
        mx = sv[0]
        for p in range(1, m):
            mx = jnp.maximum(mx, sv[p])
        ev = [jnp.exp(sv[p] - mx) for p in range(m)]
        denom = ev[0]
        for p in range(1, m):
            denom = denom + ev[p]
        acc = parts[0] * ev[0]
        for p in range(1, m):
            acc = acc + parts[p] * ev[p]
        out_ref[...] = acc / denom

    return pl.pallas_call(
        body_b,
        grid=(g,),
        in_specs=[rowh] + sess_specs + [smem_spec],
        out_specs=row64,
        out_shape=jax.ShapeDtypeStruct((n, d), f32),
    )(hcat, *sess_in, wsum)


def kernel(item, locs, times, session_emb, params, Hg):
    iids = item.reshape(-1)

    def tf(a):
        return a.T.reshape(-1)

    (self_i, mii, mti, mli, self_l, mil, mtl, self_t, mit, mlt) = \
        _sc_all(params['item_tab'], params['loc_tab'],
                params['time_tab'], iids, locs, times,
                tf(Hg['II']), tf(Hg['TI']), tf(Hg['LI']), tf(Hg['IL']),
                tf(Hg['TL']), tf(Hg['IT']), tf(Hg['LT']))
    sess = session_emb[:, 0, :]

    def b2(name):
        return params[name].reshape(1, D)

    h_items = _att_head(
        self_i, [mii, mti, mli],
        [params['II_W'], params['TI_W'], params['LI_W']],
        [b2('II_b'), b2('TI_b'), b2('LI_b')],
        None, params['att_i_W1'], b2('att_i_b1'), params['att_i_q'])
    h_locs = _att_head(
        self_l, [mil, mtl],
        [params['IL_W'], params['TL_W']],
        [b2('IL_b'), b2('TL_b')],
        sess, params['att_l_W1'], b2('att_l_b1'), params['att_l_q'])
    h_times = _att_head(
        self_t, [mit, mlt],
        [params['IT_W'], params['LT_W']],
        [b2('IT_b'), b2('LT_b')],
        sess, params['att_t_W1'], b2('att_t_b1'), params['att_t_q'])
    return (h_items, h_locs, h_times)


# speedup vs baseline: 12.5056x; 1.1139x over previous
import functools

import jax
import jax.numpy as jnp
from jax import lax
from jax.experimental import pallas as pl
from jax.experimental.pallas import tpu as pltpu
from jax.experimental.pallas import tpu_sc as plsc

D = 64
K = 16
CHUNK = 32
FLAT = CHUNK * K
SUB = 512
NC = 2
NS = 16
NW = NC * NS


IDXC = 80


def _mesh():
    return plsc.VectorSubcoreMesh(
        core_axis_name="c", subcore_axis_name="s", num_cores=NC,
        num_subcores=NS)


def _sc_all(item_tab, loc_tab, time_tab, iids, locs, times,
            g_ii, g_ti, g_li, g_il, g_tl, g_it, g_lt):
    nb = iids.shape[0]
    ns = locs.shape[0]
    rows_w = nb // NW
    f32 = jnp.float32
    i32 = jnp.int32

    out_type = tuple(
        jax.ShapeDtypeStruct(s, f32)
        for s in [(nb, D), (nb, D), (nb, D), (nb, D),
                  (ns, D), (ns, D), (ns, D),
                  (ns, D), (ns, D), (ns, D)]
    )
    scratch = [
        pltpu.VMEM((rows_w,), i32),
        pltpu.VMEM((CHUNK,), i32),
        pltpu.VMEM((CHUNK,), i32),
        pltpu.VMEM((IDXC, D), f32),
        pltpu.VMEM((IDXC, D), f32),
        pltpu.VMEM((FLAT,), i32),
        pltpu.VMEM((FLAT,), i32),
        pltpu.VMEM((FLAT,), i32),
        pltpu.VMEM((FLAT,), i32),
        pltpu.VMEM((FLAT, D), f32),
        pltpu.VMEM((FLAT, D), f32),
        pltpu.VMEM((CHUNK, D), f32),
        pltpu.VMEM((CHUNK, D), f32),
        pltpu.SemaphoreType.DMA,
        pltpu.SemaphoreType.DMA,
        pltpu.SemaphoreType.DMA,
        pltpu.SemaphoreType.DMA,
        pltpu.SemaphoreType.DMA,
        pltpu.SemaphoreType.DMA,
    ]

    @functools.partial(
        pl.kernel, out_type=out_type, mesh=_mesh(), scratch_types=scratch,
        compiler_params=pltpu.CompilerParams(use_tc_tiling_on_sc=False))
    def sc(item_t, loc_t, time_t, iids_h, locs_h, times_h,
           hii, hti, hli, hil, htl, hit, hlt,
           o_self_i, o_mii, o_mti, o_mli,
           o_self_l, o_mil, o_mtl,
           o_self_t, o_mit, o_mlt,
           aidx, lidx, tidx, srows0, srows1, fidx0, fidx1, nbr0, nbr1,
           rows0, rows1, mbuf0, mbuf1,
           nsem0, nsem1, esem0, esem1, osem0, osem1):
        wid = lax.axis_index("s") * NC + lax.axis_index("c")
        srows = (srows0, srows1)
        fidx = (fidx0, fidx1)
        nbr = (nbr0, nbr1)
        rows_b = (rows0, rows1)
        mbuf = (mbuf0, mbuf1)
        nsems = (nsem0, nsem1)
        esems = (esem0, esem1)
        osems = (osem0, osem1)

        pltpu.sync_copy(iids_h.at[pl.ds(wid * rows_w, rows_w)], aidx)
        pltpu.sync_copy(locs_h.at[pl.ds(wid * CHUNK, CHUNK)], lidx)
        pltpu.sync_copy(times_h.at[pl.ds(wid * CHUNK, CHUNK)], tidx)

        def reduce_mean(rows, mb):
            def body(c, carry):
                for d in range(D // 16):
                    sl = pl.ds(d * 16, 16)
                    acc = rows[c, sl]
                    for kk in range(1, K):
                        acc = acc + rows[kk * CHUNK + c, sl]
                    mb[c, sl] = acc * (1.0 / K)
                return carry
            lax.fori_loop(0, CHUNK, body, 0)

        def self_job(idx_all, n, tab, out):
            per_w = n // NW
            c = min(IDXC, per_w)
            nch = per_w // c

            def start(j, b):
                pltpu.async_copy(tab.at[idx_all.at[pl.ds(j * c, c)]],
                                 srows[b].at[pl.ds(0, c)], esems[b])

            def finish(j, b):
                pltpu.make_async_copy(tab.at[idx_all.at[pl.ds(j * c, c)]],
                                      srows[b].at[pl.ds(0, c)],
                                      esems[b]).wait()
                pltpu.sync_copy(srows[b].at[pl.ds(0, c)],
                                out.at[pl.ds(wid * per_w + j * c, c)])

            if nch == 1:
                start(0, 0)
                finish(0, 0)
                return
            start(0, 0)
            start(1, 1)

            def outer(jj, carry):
                for b in (0, 1):
                    j = jj * 2 + b
                    finish(j, b)
                    nxt = j + 2

                    @pl.when(nxt < nch)
                    def _():
                        start(nxt, b)
                return carry
            lax.fori_loop(0, nch // 2, outer, 0)

        def mean_job(hgf, mult, idx_all, n, tab, out):
            per_w = n // NW
            nch = per_w // CHUNK

            def start(j, b):
                fx = fidx[b]
                v0 = idx_all[pl.ds(j * CHUNK, 16)]
                v1 = idx_all[pl.ds(j * CHUNK + 16, 16)]
                for kk in range(K):
                    fx[pl.ds(kk * CHUNK, 16)] = v0 + kk * mult
                    fx[pl.ds(kk * CHUNK + 16, 16)] = v1 + kk * mult
                pltpu.async_copy(hgf.at[fx], nbr[b], nsems[b])

            def mid(j, b):
                pltpu.make_async_copy(hgf.at[fidx[b]], nbr[b],
                                      nsems[b]).wait()
                pltpu.async_copy(tab.at[nbr[b]], rows_b[b], esems[b])

            def store_wait(j, b):
                pltpu.make_async_copy(
                    mbuf[b], out.at[pl.ds(wid * per_w + j * CHUNK, CHUNK)],
                    osems[b]).wait()

            def finish(j, b, drain):
                pltpu.make_async_copy(tab.at[nbr[b]], rows_b[b],
                                      esems[b]).wait()

                @pl.when(drain)
                def _():
                    store_wait(j - 2, b)
                reduce_mean(rows_b[b], mbuf[b])
                pltpu.async_copy(
                    mbuf[b], out.at[pl.ds(wid * per_w + j * CHUNK, CHUNK)],
                    osems[b])

            if nch == 1:
                start(0, 0)
                mid(0, 0)
                finish(0, 0, jnp.bool_(False))
                store_wait(0, 0)
                return
            start(0, 0)
            start(1, 1)
            mid(0, 0)

            def outer(jj, carry):
                for b in (0, 1):
                    j = jj * 2 + b
                    pltpu.make_async_copy(tab.at[nbr[b]], rows_b[b],
                                          esems[b]).wait()

                    @pl.when(j >= 2)
                    def _():
                        store_wait(j - 2, b)

                    @pl.when(j + 2 < nch)
                    def _():
                        start(j + 2, b)

                    @pl.when(j + 1 < nch)
                    def _():
                        mid(j + 1, 1 - b)
                    reduce_mean(rows_b[b], mbuf[b])
                    pltpu.async_copy(
                        mbuf[b],
                        out.at[pl.ds(wid * per_w + j * CHUNK, CHUNK)],
                        osems[b])
                return carry
            lax.fori_loop(0, nch // 2, outer, 0)
            store_wait(nch - 2, 0)
            store_wait(nch - 1, 1)

        NI, NL, NT = item_t.shape[0], loc_t.shape[0], time_t.shape[0]
        mean_job(hii, NI, aidx, nb, item_t, o_mii)
        mean_job(hti, NI, aidx, nb, time_t, o_mti)
        mean_job(hli, NI, aidx, nb, loc_t, o_mli)
        self_job(aidx, nb, item_t, o_self_i)
        self_job(lidx, ns, loc_t, o_self_l)
        self_job(tidx, ns, time_t, o_self_t)
        mean_job(hil, NL, lidx, ns, item_t, o_mil)
        mean_job(htl, NL, lidx, ns, time_t, o_mtl)
        mean_job(hit, NT, tidx, ns, item_t, o_mit)
        mean_job(hlt, NT, tidx, ns, loc_t, o_mlt)

    return sc(item_tab, loc_tab, time_tab, iids, locs, times,
              g_ii, g_ti, g_li, g_il, g_tl, g_it, g_lt)


def _att_head(self_rows, aggs, ws, bs, sess, att_w1, att_b1, att_q):
    n, d = self_rows.shape
    n_agg = len(aggs)
    m = n_agg + (1 if sess is not None else 0)
    kx = (1 + n_agg) * d
    hw = n_agg * d
    zw = m * d
    r = min(2048, n)
    g = n // r
    f32 = jnp.float32
    inv_n = 1.0 / n

    wcat = jnp.zeros((kx, hw), f32)
    for p in range(n_agg):
        wcat = wcat.at[0:d, p * d:(p + 1) * d].set(ws[p][:d, :])
        wcat = wcat.at[(1 + p) * d:(2 + p) * d, p * d:(p + 1) * d].set(
            ws[p][d:, :])
    bcat = jnp.concatenate(bs, axis=1)
    w1d = jnp.kron(jnp.eye(m, dtype=f32), att_w1)
    b1d = jnp.tile(att_b1, (1, m))
    qd = jnp.kron(jnp.eye(m, dtype=f32), att_q)

    row64 = pl.BlockSpec((r, d), lambda i: (i, 0))
    rowh = pl.BlockSpec((r, hw), lambda i: (i, 0))
    smem_spec = pl.BlockSpec(memory_space=pltpu.SMEM)

    def full(a):
        return pl.BlockSpec(a.shape, lambda i: tuple(0 for _ in a.shape))

    sess_in = [sess] if sess is not None else []
    sess_specs = [row64] if sess is not None else []

    def body_a(*args):
        i = pl.program_id(0)
        it = iter(args)
        self_b = next(it)[...]
        agg_b = [next(it)[...] for _ in range(n_agg)]
        sess_b = next(it)[...] if sess is not None else None
        wcat_b = next(it)[...]
        bcat_b = next(it)[...]
        w1d_b = next(it)[...]
        b1d_b = next(it)[...]
        qd_b = next(it)[...]
        hcat_ref = next(it)
        wsum_ref = next(it)

        x = jnp.concatenate([self_b] + agg_b, axis=1)
        h = jnp.maximum(
            jnp.dot(x, wcat_b, preferred_element_type=f32) + bcat_b, 0.0)
        hcat_ref[...] = h
        z = h if sess_b is None else jnp.concatenate([h, sess_b], axis=1)
        t = jnp.tanh(jnp.dot(z, w1d_b, preferred_element_type=f32) + b1d_b)
        w3 = jnp.dot(t, qd_b, preferred_element_type=f32)

        @pl.when(i == 0)
        def _():
            for p in range(m):
                wsum_ref[p] = 0.0

        for p in range(m):
            wsum_ref[p] += jnp.sum(w3[:, p:p + 1])

    hcat, wsum = pl.pallas_call(
        body_a,
        grid=(g,),
        in_specs=[row64] + [row64] * n_agg + sess_specs
                 + [full(wcat), full(bcat), full(w1d), full(b1d), full(qd)],
        out_specs=(rowh, smem_spec),
        out_shape=(jax.ShapeDtypeStruct((n, hw), f32),
                   jax.ShapeDtypeStruct((m,), f32)),
    )(self_rows, *aggs, *sess_in, wcat, bcat, w1d, b1d, qd)

    def body_b(*args):
        it = iter(args)
        hcat_b = next(it)[...]
        sess_b = next(it)[...] if sess is not None else None
        wsum_ref = next(it)
        out_ref = next(it)

        parts = [hcat_b[:, p * d:(p + 1) * d] for p in range(n_agg)]
        if sess_b is not None:
            parts.append(sess_b)
        sv = [jnp.full((1, 1), wsum_ref[p] * inv_n, f32) for p in range(m)]
        mx = sv[0]
        for p in range(1, m):
            mx = jnp.maximum(mx, sv[p])
        ev = [jnp.exp(sv[p] - mx) for p in range(m)]
        denom = ev[0]
        for p in range(1, m):
            denom = denom + ev[p]
        acc = parts[0] * ev[0]
        for p in range(1, m):
            acc = acc + parts[p] * ev[p]
        out_ref[...] = acc / denom

    return pl.pallas_call(
        body_b,
        grid=(g,),
        in_specs=[rowh] + sess_specs + [smem_spec],
        out_specs=row64,
        out_shape=jax.ShapeDtypeStruct((n, d), f32),
    )(hcat, *sess_in, wsum)


def kernel(item, locs, times, session_emb, params, Hg):
    iids = item.reshape(-1)

    def tf(a):
        return a.T.reshape(-1)

    (self_i, mii, mti, mli, self_l, mil, mtl, self_t, mit, mlt) = \
        _sc_all(params['item_tab'], params['loc_tab'],
                params['time_tab'], iids, locs, times,
                tf(Hg['II']), tf(Hg['TI']), tf(Hg['LI']), tf(Hg['IL']),
                tf(Hg['TL']), tf(Hg['IT']), tf(Hg['LT']))
    sess = session_emb[:, 0, :]

    def b2(name):
        return params[name].reshape(1, D)

    h_items = _att_head(
        self_i, [mii, mti, mli],
        [params['II_W'], params['TI_W'], params['LI_W']],
        [b2('II_b'), b2('TI_b'), b2('LI_b')],
        None, params['att_i_W1'], b2('att_i_b1'), params['att_i_q'])
    h_locs = _att_head(
        self_l, [mil, mtl],
        [params['IL_W'], params['TL_W']],
        [b2('IL_b'), b2('TL_b')],
        sess, params['att_l_W1'], b2('att_l_b1'), params['att_l_q'])
    h_times = _att_head(
        self_t, [mit, mlt],
        [params['IT_W'], params['LT_W']],
        [b2('IT_b'), b2('LT_b')],
        sess, params['att_t_W1'], b2('att_t_b1'), params['att_t_q'])
    return (h_items, h_locs, h_times)
